# baseline (speedup 1.0000x reference)
import functools

import jax
import jax.numpy as jnp
from jax.experimental import pallas as pl
from jax.experimental.pallas import tpu as pltpu

N = 4096
D = 256
V = 3
H = 64
B = 1024
A = 128
C = 4
SLOPE = 0.2

RPB = 8


def _proj_kernel(feat_ref, w_ref, b_ref, h_ref):
    h_ref[0] = (
        jnp.dot(feat_ref[...], w_ref[0], preferred_element_type=jnp.float32)
        + b_ref[0]
    )


def _agg_kernel(idx_ref, thr_ref, h_ref, *rest):
    row_refs = rest[:RPB]
    out_ref = rest[RPB]
    v = pl.program_id(0)
    b = pl.program_id(1)
    thr = thr_ref[0, 0, 0]
    rows = jnp.concatenate([r[0] for r in row_refs], axis=0)
    rows_m = rows * (rows >= thr).astype(jnp.float32)
    deg = jnp.maximum(jnp.sum(rows_m, axis=1, keepdims=True), 1e-12)
    rn = rows_m / deg
    col = jnp.concatenate(
        [jnp.full((1, 1), idx_ref[b * RPB + j], jnp.int32) for j in range(RPB)],
        axis=0,
    )
    iota = jax.lax.broadcasted_iota(jnp.int32, (RPB, N), 1)
    aug = rn + (iota == col).astype(jnp.float32)
    acc = jnp.dot(aug, h_ref[0], preferred_element_type=jnp.float32)
    out_ref[0] = jnp.where(acc >= 0, acc, SLOPE * acc)


def _attn_kernel(stack_ref, wa_ref, ba_ref, va_ref, wo_ref, bo_ref,
                 bf_ref, pred_ref):
    stk = stack_ref[...]
    wa = wa_ref[...]
    ba = ba_ref[...]
    va = va_ref[...]
    es = []
    for v in range(V):
        s = jnp.tanh(jnp.dot(stk[v], wa, preferred_element_type=jnp.float32) + ba)
        es.append(jnp.sum(s * va) / B)
    m = jnp.maximum(es[0], jnp.maximum(es[1], es[2]))
    ws = [jnp.exp(e - m) for e in es]
    tot = ws[0] + ws[1] + ws[2]
    bf = (ws[0] * stk[0] + ws[1] * stk[1] + ws[2] * stk[2]) / tot
    bf_ref[...] = bf
    pred_ref[...] = (
        jnp.dot(bf, wo_ref[...], preferred_element_type=jnp.float32) + bo_ref[...]
    )


@jax.jit
def kernel(features, weights, batch_idx, thresholds, W_intra, b_intra,
           W_attn, b_attn, v_attn, W_out, b_out):
    batch_idx = batch_idx.astype(jnp.int32)

    h = pl.pallas_call(
        _proj_kernel,
        grid=(V,),
        in_specs=[
            pl.BlockSpec((N, D), lambda v: (0, 0)),
            pl.BlockSpec((1, D, H), lambda v: (v, 0, 0)),
            pl.BlockSpec((1, 1, H), lambda v: (v, 0, 0)),
        ],
        out_specs=pl.BlockSpec((1, N, H), lambda v: (v, 0, 0)),
        out_shape=jax.ShapeDtypeStruct((V, N, H), jnp.float32),
    )(features, W_intra, b_intra.reshape(V, 1, H))

    weights_r = weights.reshape(V * N, 1, N)
    thr_r = thresholds.reshape(V, 1, 1)

    def row_spec(j):
        return pl.BlockSpec(
            (1, 1, N),
            lambda v, b, idx: (v * N + idx[b * RPB + j], 0, 0),
        )

    grid_spec = pltpu.PrefetchScalarGridSpec(
        num_scalar_prefetch=1,
        grid=(V, B // RPB),
        in_specs=[
            pl.BlockSpec((1, 1, 1), lambda v, b, idx: (v, 0, 0)),
            pl.BlockSpec((1, N, H), lambda v, b, idx: (v, 0, 0)),
        ] + [row_spec(j) for j in range(RPB)],
        out_specs=pl.BlockSpec((1, RPB, H), lambda v, b, idx: (v, b, 0)),
    )
    stack = pl.pallas_call(
        _agg_kernel,
        grid_spec=grid_spec,
        out_shape=jax.ShapeDtypeStruct((V, B, H), jnp.float32),
    )(batch_idx, thr_r, h, *([weights_r] * RPB))

    bf, pred = pl.pallas_call(
        _attn_kernel,
        in_specs=[
            pl.BlockSpec((V, B, H), lambda: (0, 0, 0)),
            pl.BlockSpec((H, A), lambda: (0, 0)),
            pl.BlockSpec((1, A), lambda: (0, 0)),
            pl.BlockSpec((1, A), lambda: (0, 0)),
            pl.BlockSpec((H, C), lambda: (0, 0)),
            pl.BlockSpec((1, C), lambda: (0, 0)),
        ],
        out_specs=[
            pl.BlockSpec((B, H), lambda: (0, 0)),
            pl.BlockSpec((B, C), lambda: (0, 0)),
        ],
        out_shape=[
            jax.ShapeDtypeStruct((B, H), jnp.float32),
            jax.ShapeDtypeStruct((B, C), jnp.float32),
        ],
    )(stack, W_attn, b_attn.reshape(1, A), v_attn.reshape(1, A),
      W_out, b_out.reshape(1, C))

    return (bf, pred)


# speedup vs baseline: 9.8406x; 9.8406x over previous
import functools

import jax
import jax.numpy as jnp
from jax.experimental import pallas as pl
from jax.experimental.pallas import tpu as pltpu

N = 4096
D = 256
V = 3
H = 64
B = 1024
A = 128
C = 4
SLOPE = 0.2

RPB = 128
NB_B = B // RPB
NSTEPS = V * NB_B
NBUF = 3


def _proj_kernel(feat_ref, w_ref, b_ref, h_ref):
    h_ref[0] = (
        jnp.dot(feat_ref[...], w_ref[0], preferred_element_type=jnp.float32)
        + b_ref[0]
    ).astype(jnp.bfloat16)


def _agg_kernel(idx_ref, thr_ref, h_ref, col_ref, w_ref, out_ref, buf, sems):
    v = pl.program_id(0)
    b = pl.program_id(1)
    step = v * NB_B + b

    def issue(t, slot):
        tv = t // NB_B
        tb = t % NB_B

        def body(j, _):
            row = tv * N + idx_ref[tb * RPB + j]
            pltpu.make_async_copy(
                w_ref.at[pl.ds(row, 1), :],
                buf.at[slot, pl.ds(j, 1), :],
                sems.at[slot],
            ).start()
            return 0

        jax.lax.fori_loop(0, RPB, body, 0)

    def drain(t, slot):
        tv = t // NB_B
        tb = t % NB_B

        def body(j, _):
            row = tv * N + idx_ref[tb * RPB + j]
            pltpu.make_async_copy(
                w_ref.at[pl.ds(row, 1), :],
                buf.at[slot, pl.ds(j, 1), :],
                sems.at[slot],
            ).wait()
            return 0

        jax.lax.fori_loop(0, RPB, body, 0)

    @pl.when(step == 0)
    def _():
        for t in range(NBUF - 1):
            issue(t, t % NBUF)

    @pl.when(step + NBUF - 1 < NSTEPS)
    def _():
        issue(step + NBUF - 1, (step + NBUF - 1) % NBUF)

    slot = jax.lax.rem(step, NBUF)
    drain(step, slot)

    rows = buf[slot]
    thr = thr_ref[0, 0, 0]
    rows_m = jnp.where(rows >= thr, rows, 0.0)
    deg = jnp.maximum(jnp.sum(rows_m, axis=1, keepdims=True), 1e-12)
    col = col_ref[0]
    iota = jax.lax.broadcasted_iota(jnp.int32, (RPB, N), 1)
    aug = (rows_m + jnp.where(iota == col, deg, 0.0)).astype(jnp.bfloat16)
    acc = jnp.dot(aug, h_ref[0], preferred_element_type=jnp.float32)
    acc = acc / deg
    out_ref[0] = jnp.where(acc >= 0, acc, SLOPE * acc)


def _attn_kernel(stack_ref, wa_ref, ba_ref, va_ref, wo_ref, bo_ref,
                 bf_ref, pred_ref):
    stk = stack_ref[...]
    wa = wa_ref[...]
    ba = ba_ref[...]
    va = va_ref[...]
    es = []
    for v in range(V):
        s = jnp.tanh(jnp.dot(stk[v], wa, preferred_element_type=jnp.float32) + ba)
        es.append(jnp.sum(s * va) / B)
    m = jnp.maximum(es[0], jnp.maximum(es[1], es[2]))
    ws = [jnp.exp(e - m) for e in es]
    tot = ws[0] + ws[1] + ws[2]
    bf = (ws[0] * stk[0] + ws[1] * stk[1] + ws[2] * stk[2]) / tot
    bf_ref[...] = bf
    pred_ref[...] = (
        jnp.dot(bf, wo_ref[...], preferred_element_type=jnp.float32) + bo_ref[...]
    )


@jax.jit
def kernel(features, weights, batch_idx, thresholds, W_intra, b_intra,
           W_attn, b_attn, v_attn, W_out, b_out):
    batch_idx = batch_idx.astype(jnp.int32)

    h = pl.pallas_call(
        _proj_kernel,
        grid=(V,),
        in_specs=[
            pl.BlockSpec((N, D), lambda v: (0, 0)),
            pl.BlockSpec((1, D, H), lambda v: (v, 0, 0)),
            pl.BlockSpec((1, 1, H), lambda v: (v, 0, 0)),
        ],
        out_specs=pl.BlockSpec((1, N, H), lambda v: (v, 0, 0)),
        out_shape=jax.ShapeDtypeStruct((V, N, H), jnp.bfloat16),
    )(features, W_intra, b_intra.reshape(V, 1, H))

    weights_2d = weights.reshape(V * N, N)
    thr_r = thresholds.reshape(V, 1, 1)
    col_r = batch_idx.reshape(NB_B, RPB, 1)

    grid_spec = pltpu.PrefetchScalarGridSpec(
        num_scalar_prefetch=1,
        grid=(V, NB_B),
        in_specs=[
            pl.BlockSpec((1, 1, 1), lambda v, b, idx: (v, 0, 0)),
            pl.BlockSpec((1, N, H), lambda v, b, idx: (v, 0, 0)),
            pl.BlockSpec((1, RPB, 1), lambda v, b, idx: (b, 0, 0)),
            pl.BlockSpec(memory_space=pl.ANY),
        ],
        out_specs=pl.BlockSpec((1, RPB, H), lambda v, b, idx: (v, b, 0)),
        scratch_shapes=[
            pltpu.VMEM((NBUF, RPB, N), jnp.float32),
            pltpu.SemaphoreType.DMA((NBUF,)),
        ],
    )
    stack = pl.pallas_call(
        _agg_kernel,
        grid_spec=grid_spec,
        out_shape=jax.ShapeDtypeStruct((V, B, H), jnp.float32),
    )(batch_idx, thr_r, h, col_r, weights_2d)

    bf, pred = pl.pallas_call(
        _attn_kernel,
        in_specs=[
            pl.BlockSpec((V, B, H), lambda: (0, 0, 0)),
            pl.BlockSpec((H, A), lambda: (0, 0)),
            pl.BlockSpec((1, A), lambda: (0, 0)),
            pl.BlockSpec((1, A), lambda: (0, 0)),
            pl.BlockSpec((H, C), lambda: (0, 0)),
            pl.BlockSpec((1, C), lambda: (0, 0)),
        ],
        out_specs=[
            pl.BlockSpec((B, H), lambda: (0, 0)),
            pl.BlockSpec((B, C), lambda: (0, 0)),
        ],
        out_shape=[
            jax.ShapeDtypeStruct((B, H), jnp.float32),
            jax.ShapeDtypeStruct((B, C), jnp.float32),
        ],
    )(stack, W_attn, b_attn.reshape(1, A), v_attn.reshape(1, A),
      W_out, b_out.reshape(1, C))

    return (bf, pred)


# speedup vs baseline: 17.0494x; 1.7326x over previous
import functools

import jax
import jax.numpy as jnp
from jax.experimental import pallas as pl
from jax.experimental.pallas import tpu as pltpu

N = 4096
D = 256
V = 3
H = 64
B = 1024
A = 128
C = 4
SLOPE = 0.2

RPB = 128
NB_B = B // RPB
NSTEPS = V * NB_B
NBUF = 3


def _proj_kernel(feat_ref, w_ref, b_ref, h_ref):
    h_ref[0] = (
        jnp.dot(feat_ref[...], w_ref[0], preferred_element_type=jnp.float32)
        + b_ref[0]
    ).astype(jnp.bfloat16)


def _agg_kernel(idx_ref, thr_ref, h_ref, col_ref, w_ref, out_ref, buf, sems):
    v = pl.program_id(0)
    b = pl.program_id(1)
    step = v * NB_B + b

    def issue(t, slot):
        tv = t // NB_B
        tb = t % NB_B
        for j in range(RPB):
            row = tv * N + idx_ref[tb * RPB + j]
            pltpu.make_async_copy(
                w_ref.at[pl.ds(row, 1), :],
                buf.at[slot, pl.ds(j, 1), :],
                sems.at[slot],
            ).start()

    def drain(slot):
        pltpu.make_async_copy(
            w_ref.at[pl.ds(0, RPB), :],
            buf.at[slot],
            sems.at[slot],
        ).wait()

    @pl.when(step == 0)
    def _():
        for t in range(NBUF - 1):
            issue(t, t % NBUF)

    @pl.when(step + NBUF - 1 < NSTEPS)
    def _():
        issue(step + NBUF - 1, (step + NBUF - 1) % NBUF)

    slot = jax.lax.rem(step, NBUF)
    drain(slot)

    rows = buf[slot]
    thr = thr_ref[0, 0, 0]
    rows_m = jnp.where(rows >= thr, rows, 0.0)
    deg = jnp.maximum(jnp.sum(rows_m, axis=1, keepdims=True), 1e-12)
    col = col_ref[0]
    iota = jax.lax.broadcasted_iota(jnp.int32, (RPB, N), 1)
    aug = (rows_m + jnp.where(iota == col, deg, 0.0)).astype(jnp.bfloat16)
    acc = jnp.dot(aug, h_ref[0], preferred_element_type=jnp.float32)
    acc = acc / deg
    out_ref[0] = jnp.where(acc >= 0, acc, SLOPE * acc)


def _attn_kernel(stack_ref, wa_ref, ba_ref, va_ref, wo_ref, bo_ref,
                 bf_ref, pred_ref):
    stk = stack_ref[...]
    wa = wa_ref[...]
    ba = ba_ref[...]
    va = va_ref[...]
    es = []
    for v in range(V):
        s = jnp.tanh(jnp.dot(stk[v], wa, preferred_element_type=jnp.float32) + ba)
        es.append(jnp.sum(s * va) / B)
    m = jnp.maximum(es[0], jnp.maximum(es[1], es[2]))
    ws = [jnp.exp(e - m) for e in es]
    tot = ws[0] + ws[1] + ws[2]
    bf = (ws[0] * stk[0] + ws[1] * stk[1] + ws[2] * stk[2]) / tot
    bf_ref[...] = bf
    pred_ref[...] = (
        jnp.dot(bf, wo_ref[...], preferred_element_type=jnp.float32) + bo_ref[...]
    )


@jax.jit
def kernel(features, weights, batch_idx, thresholds, W_intra, b_intra,
           W_attn, b_attn, v_attn, W_out, b_out):
    batch_idx = batch_idx.astype(jnp.int32)

    h = pl.pallas_call(
        _proj_kernel,
        grid=(V,),
        in_specs=[
            pl.BlockSpec((N, D), lambda v: (0, 0)),
            pl.BlockSpec((1, D, H), lambda v: (v, 0, 0)),
            pl.BlockSpec((1, 1, H), lambda v: (v, 0, 0)),
        ],
        out_specs=pl.BlockSpec((1, N, H), lambda v: (v, 0, 0)),
        out_shape=jax.ShapeDtypeStruct((V, N, H), jnp.bfloat16),
    )(features, W_intra, b_intra.reshape(V, 1, H))

    weights_2d = weights.reshape(V * N, N)
    thr_r = thresholds.reshape(V, 1, 1)
    col_r = batch_idx.reshape(NB_B, RPB, 1)

    grid_spec = pltpu.PrefetchScalarGridSpec(
        num_scalar_prefetch=1,
        grid=(V, NB_B),
        in_specs=[
            pl.BlockSpec((1, 1, 1), lambda v, b, idx: (v, 0, 0)),
            pl.BlockSpec((1, N, H), lambda v, b, idx: (v, 0, 0)),
            pl.BlockSpec((1, RPB, 1), lambda v, b, idx: (b, 0, 0)),
            pl.BlockSpec(memory_space=pl.ANY),
        ],
        out_specs=pl.BlockSpec((1, RPB, H), lambda v, b, idx: (v, b, 0)),
        scratch_shapes=[
            pltpu.VMEM((NBUF, RPB, N), jnp.float32),
            pltpu.SemaphoreType.DMA((NBUF,)),
        ],
    )
    stack = pl.pallas_call(
        _agg_kernel,
        grid_spec=grid_spec,
        out_shape=jax.ShapeDtypeStruct((V, B, H), jnp.float32),
    )(batch_idx, thr_r, h, col_r, weights_2d)

    bf, pred = pl.pallas_call(
        _attn_kernel,
        in_specs=[
            pl.BlockSpec((V, B, H), lambda: (0, 0, 0)),
            pl.BlockSpec((H, A), lambda: (0, 0)),
            pl.BlockSpec((1, A), lambda: (0, 0)),
            pl.BlockSpec((1, A), lambda: (0, 0)),
            pl.BlockSpec((H, C), lambda: (0, 0)),
            pl.BlockSpec((1, C), lambda: (0, 0)),
        ],
        out_specs=[
            pl.BlockSpec((B, H), lambda: (0, 0)),
            pl.BlockSpec((B, C), lambda: (0, 0)),
        ],
        out_shape=[
            jax.ShapeDtypeStruct((B, H), jnp.float32),
            jax.ShapeDtypeStruct((B, C), jnp.float32),
        ],
    )(stack, W_attn, b_attn.reshape(1, A), v_attn.reshape(1, A),
      W_out, b_out.reshape(1, C))

    return (bf, pred)


# speedup vs baseline: 17.9273x; 1.0515x over previous
import jax
import jax.numpy as jnp
from jax.experimental import pallas as pl
from jax.experimental.pallas import tpu as pltpu

N = 4096
D = 256
V = 3
H = 64
B = 1024
A = 128
C = 4
SLOPE = 0.2

RPB = 128
NB_B = B // RPB
NSTEPS = V * NB_B
NBUF = 3


def _fused_kernel(idx_ref, thr_ref, col_ref, feat_ref, wi_ref, bi_ref,
                  wa_ref, ba_ref, va_ref, wo_ref, bo_ref, w_ref,
                  bf_ref, pred_ref, buf, sems, h_buf, stack_buf):
    v = pl.program_id(0)
    b = pl.program_id(1)
    step = v * NB_B + b

    def issue(t, slot):
        tv = t // NB_B
        tb = t % NB_B
        for j in range(RPB):
            row = tv * N + idx_ref[tb * RPB + j]
            pltpu.make_async_copy(
                w_ref.at[pl.ds(row, 1), :],
                buf.at[slot, pl.ds(j, 1), :],
                sems.at[slot],
            ).start()

    @pl.when(step == 0)
    def _():
        for t in range(NBUF - 1):
            issue(t, t % NBUF)

    @pl.when(step + NBUF - 1 < NSTEPS)
    def _():
        issue(step + NBUF - 1, (step + NBUF - 1) % NBUF)

    @pl.when(b == 0)
    def _():
        h_buf[...] = (
            jnp.dot(feat_ref[...], wi_ref[0], preferred_element_type=jnp.float32)
            + bi_ref[0]
        ).astype(jnp.bfloat16)

    slot = jax.lax.rem(step, NBUF)
    pltpu.make_async_copy(
        w_ref.at[pl.ds(0, RPB), :],
        buf.at[slot],
        sems.at[slot],
    ).wait()

    rows = buf[slot]
    thr = thr_ref[0, 0, 0]
    rows_m = jnp.where(rows >= thr, rows, 0.0)
    deg = jnp.maximum(jnp.sum(rows_m, axis=1, keepdims=True), 1e-12)
    col = col_ref[0]
    iota = jax.lax.broadcasted_iota(jnp.int32, (RPB, N), 1)
    aug = (rows_m + jnp.where(iota == col, deg, 0.0)).astype(jnp.bfloat16)
    acc = jnp.dot(aug, h_buf[...], preferred_element_type=jnp.float32)
    acc = acc / deg
    stack_buf[v, pl.ds(b * RPB, RPB), :] = jnp.where(acc >= 0, acc, SLOPE * acc)

    @pl.when(step == NSTEPS - 1)
    def _():
        stk = stack_buf[...]
        wa = wa_ref[...]
        ba = ba_ref[...]
        va = va_ref[...]
        es = []
        for i in range(V):
            s = jnp.tanh(
                jnp.dot(stk[i], wa, preferred_element_type=jnp.float32) + ba
            )
            es.append(jnp.sum(s * va) / B)
        m = jnp.maximum(es[0], jnp.maximum(es[1], es[2]))
        ws = [jnp.exp(e - m) for e in es]
        tot = ws[0] + ws[1] + ws[2]
        bf = (ws[0] * stk[0] + ws[1] * stk[1] + ws[2] * stk[2]) / tot
        bf_ref[...] = bf
        pred_ref[...] = (
            jnp.dot(bf, wo_ref[...], preferred_element_type=jnp.float32)
            + bo_ref[...]
        )


@jax.jit
def kernel(features, weights, batch_idx, thresholds, W_intra, b_intra,
           W_attn, b_attn, v_attn, W_out, b_out):
    batch_idx = batch_idx.astype(jnp.int32)
    weights_2d = weights.reshape(V * N, N)
    thr_r = thresholds.reshape(V, 1, 1)
    col_r = batch_idx.reshape(NB_B, RPB, 1)

    grid_spec = pltpu.PrefetchScalarGridSpec(
        num_scalar_prefetch=1,
        grid=(V, NB_B),
        in_specs=[
            pl.BlockSpec((1, 1, 1), lambda v, b, idx: (v, 0, 0)),
            pl.BlockSpec((1, RPB, 1), lambda v, b, idx: (b, 0, 0)),
            pl.BlockSpec((N, D), lambda v, b, idx: (0, 0)),
            pl.BlockSpec((1, D, H), lambda v, b, idx: (v, 0, 0)),
            pl.BlockSpec((1, 1, H), lambda v, b, idx: (v, 0, 0)),
            pl.BlockSpec((H, A), lambda v, b, idx: (0, 0)),
            pl.BlockSpec((1, A), lambda v, b, idx: (0, 0)),
            pl.BlockSpec((1, A), lambda v, b, idx: (0, 0)),
            pl.BlockSpec((H, C), lambda v, b, idx: (0, 0)),
            pl.BlockSpec((1, C), lambda v, b, idx: (0, 0)),
            pl.BlockSpec(memory_space=pl.ANY),
        ],
        out_specs=[
            pl.BlockSpec((B, H), lambda v, b, idx: (0, 0)),
            pl.BlockSpec((B, C), lambda v, b, idx: (0, 0)),
        ],
        scratch_shapes=[
            pltpu.VMEM((NBUF, RPB, N), jnp.float32),
            pltpu.SemaphoreType.DMA((NBUF,)),
            pltpu.VMEM((N, H), jnp.bfloat16),
            pltpu.VMEM((V, B, H), jnp.float32),
        ],
    )
    bf, pred = pl.pallas_call(
        _fused_kernel,
        grid_spec=grid_spec,
        out_shape=[
            jax.ShapeDtypeStruct((B, H), jnp.float32),
            jax.ShapeDtypeStruct((B, C), jnp.float32),
        ],
    )(batch_idx, thr_r, col_r, features, W_intra, b_intra.reshape(V, 1, H),
      W_attn, b_attn.reshape(1, A), v_attn.reshape(1, A),
      W_out, b_out.reshape(1, C), weights_2d)

    return (bf, pred)
